# baseline (speedup 1.0000x reference)
import functools

import jax
import jax.numpy as jnp
from jax import lax
from jax.experimental import pallas as pl
from jax.experimental.pallas import tpu as pltpu
from jax.experimental.pallas import tpu_sc as plsc

N = 10000
E = 320000
D = 128
R = 8
T = 64
NTILES = 32
NPAD = 10240
CHUNK = 64
N_CHUNKS = 160
NC0 = 80
NC1 = 2 * N_CHUNKS - NC0
EPT = N_CHUNKS * CHUNK
E_PAD = NTILES * EPT
ROWS_PER_SUB = NPAD // 16
QC = 16
NBUF = 4
LANES = 16

_MESH = plsc.VectorSubcoreMesh(core_axis_name="c", subcore_axis_name="s")


def _sc_edge_pass(msg_width, use_nid_table):
    stride = N if use_nid_table else NPAD

    def body(*refs):
        if use_nid_table:
            (nid_hbm, src_hbm, dst_hbm, ety_hbm, tab_hbm, zrow_hbm, zcnt_hbm,
             acc_hbm, cnt_hbm,
             srcbuf, etybuf, dstbuf, rowbuf, mb0, mb1, mb2, mb3, onesbuf,
             acc_sh, cnt_sh, sem_a, sg0, sg1, sg2, sg3, ss0, ss1, ss2,
             ss3) = refs
        else:
            (src_hbm, dst_hbm, ety_hbm, tab_hbm, zrow_hbm,
             acc_hbm,
             srcbuf, etybuf, dstbuf, rowbuf, mb0, mb1, mb2, mb3,
             acc_sh, sem_a, sg0, sg1, sg2, sg3, ss0, ss1, ss2, ss3) = refs
        c = lax.axis_index("c")
        s = lax.axis_index("s")
        gwid = c * 16 + s

        pltpu.sync_copy(zrow_hbm, acc_sh.at[pl.ds(s * ROWS_PER_SUB, ROWS_PER_SUB)])
        if use_nid_table:
            pltpu.sync_copy(zcnt_hbm, cnt_sh.at[pl.ds(s * ROWS_PER_SUB, ROWS_PER_SUB)])
            for i in range(CHUNK // LANES):
                onesbuf[pl.ds(i * LANES, LANES)] = jnp.ones((LANES,), jnp.float32)
        plsc.subcore_barrier()

        bufs = (mb0, mb1, mb2, mb3)
        gsems = (sg0, sg1, sg2, sg3)
        ssems = (ss0, ss1, ss2, ss3)
        nsp = jnp.where(c == 0, NC0 // QC, NC1 // QC)
        base_chunk = jnp.where(c == 0, s * NC0, 16 * NC0 + s * NC1)

        def subpass(q, carry0):
            qb = pl.ds(base_chunk + q * QC, QC)
            cps = [pltpu.async_copy(src_hbm.at[qb], srcbuf, sem_a),
                   pltpu.async_copy(ety_hbm.at[qb], etybuf, sem_a),
                   pltpu.async_copy(dst_hbm.at[qb], dstbuf, sem_a)]
            for cp in cps:
                cp.wait()
            if use_nid_table:
                def nid_fire(j, carry):
                    pltpu.async_copy(nid_hbm.at[srcbuf.at[j]], rowbuf.at[j],
                                     sem_a)
                    return carry

                lax.fori_loop(0, QC, nid_fire, 0)

                def nid_drain(j, carry):
                    pltpu.make_async_copy(nid_hbm.at[srcbuf.at[0]],
                                          rowbuf.at[0], sem_a).wait()
                    return carry

                lax.fori_loop(0, QC, nid_drain, 0)

            def rowcalc(j, carry):
                for i in range(CHUNK // LANES):
                    sl = pl.ds(i * LANES, LANES)
                    s16 = rowbuf[j, sl] if use_nid_table else srcbuf[j, sl]
                    rowbuf[j, sl] = etybuf[j, sl] * stride + s16
                return carry

            lax.fori_loop(0, QC, rowcalc, 0)

            for b in range(NBUF):
                pltpu.async_copy(tab_hbm.at[rowbuf.at[b]], bufs[b], gsems[b])

            def chunk_quad(jj, carry):
                for b in range(NBUF):
                    j = NBUF * jj + b
                    pltpu.make_async_copy(tab_hbm.at[rowbuf.at[0]], bufs[b],
                                          gsems[b]).wait()
                    cp_m = pltpu.async_copy(bufs[b], acc_sh.at[dstbuf.at[j]],
                                            ssems[b], add=True)
                    if use_nid_table:
                        cp_c = pltpu.async_copy(onesbuf,
                                                cnt_sh.at[dstbuf.at[j]],
                                                ssems[b], add=True)
                    cp_m.wait()
                    if use_nid_table:
                        cp_c.wait()

                    @pl.when(jj < QC // NBUF - 1)
                    def _():
                        pltpu.async_copy(tab_hbm.at[rowbuf.at[j + NBUF]],
                                         bufs[b], gsems[b])
                return carry

            lax.fori_loop(0, QC // NBUF, chunk_quad, 0)
            return carry0

        lax.fori_loop(0, nsp, subpass, 0)
        plsc.subcore_barrier()

        rows = pl.ds(s * ROWS_PER_SUB, ROWS_PER_SUB)
        pltpu.sync_copy(acc_sh.at[rows], acc_hbm.at[c, rows])
        if use_nid_table:
            pltpu.sync_copy(cnt_sh.at[rows], cnt_hbm.at[c, rows])

    if use_nid_table:
        out_type = (jax.ShapeDtypeStruct((2, NPAD, msg_width), jnp.float32),
                    jax.ShapeDtypeStruct((2, NPAD), jnp.float32))
        scratch = [
            pltpu.VMEM((QC, CHUNK), jnp.int32),
            pltpu.VMEM((QC, CHUNK), jnp.int32),
            pltpu.VMEM((QC, CHUNK), jnp.int32),
            pltpu.VMEM((QC, CHUNK), jnp.int32),
            pltpu.VMEM((CHUNK, msg_width), jnp.float32),
            pltpu.VMEM((CHUNK, msg_width), jnp.float32),
            pltpu.VMEM((CHUNK, msg_width), jnp.float32),
            pltpu.VMEM((CHUNK, msg_width), jnp.float32),
            pltpu.VMEM((CHUNK,), jnp.float32),
            pltpu.VMEM_SHARED((NPAD, msg_width), jnp.float32),
            pltpu.VMEM_SHARED((NPAD,), jnp.float32),
        ] + [pltpu.SemaphoreType.DMA] * 9
    else:
        out_type = jax.ShapeDtypeStruct((2, NPAD, msg_width), jnp.float32)
        scratch = [
            pltpu.VMEM((QC, CHUNK), jnp.int32),
            pltpu.VMEM((QC, CHUNK), jnp.int32),
            pltpu.VMEM((QC, CHUNK), jnp.int32),
            pltpu.VMEM((QC, CHUNK), jnp.int32),
            pltpu.VMEM((CHUNK, msg_width), jnp.float32),
            pltpu.VMEM((CHUNK, msg_width), jnp.float32),
            pltpu.VMEM((CHUNK, msg_width), jnp.float32),
            pltpu.VMEM((CHUNK, msg_width), jnp.float32),
            pltpu.VMEM_SHARED((NPAD, msg_width), jnp.float32),
        ] + [pltpu.SemaphoreType.DMA] * 9
    return pl.kernel(body, mesh=_MESH, out_type=out_type, scratch_types=scratch)


def _dot(a, b):
    return lax.dot_general(a, b, (((1,), (0,)), ((), ())),
                           preferred_element_type=jnp.float32,
                           precision=lax.Precision.HIGHEST)


def _tc1_body(x_ref, w_ref, o_ref):
    x = x_ref[...]
    for r in range(R):
        o_ref[r] = _dot(x, w_ref[r])


def _tc2_body(a_ref, c_ref, w_ref, o_ref):
    cnt = jnp.maximum(c_ref[0] + c_ref[1], 1.0)
    h = jnp.maximum((a_ref[0] + a_ref[1]) / cnt[:, None], 0.0)
    for r in range(R):
        o_ref[r] = _dot(h, w_ref[r])


def _tc3_body(a_ref, c_ref, o_ref):
    cnt = jnp.maximum(c_ref[0] + c_ref[1], 1.0)
    o_ref[...] = ((a_ref[0] + a_ref[1]) / cnt[:, None])[:, :T]


def _tc1(entity, W1):
    bn = 2000
    return pl.pallas_call(
        _tc1_body,
        grid=(N // bn,),
        in_specs=[
            pl.BlockSpec((bn, D), lambda i: (i, 0)),
            pl.BlockSpec((R, D, D), lambda i: (0, 0, 0)),
        ],
        out_specs=pl.BlockSpec((R, bn, D), lambda i: (0, i, 0)),
        out_shape=jax.ShapeDtypeStruct((R, N, D), jnp.float32),
    )(entity, W1)


def _tc2(acc1, cnt, W2):
    bn = 2048
    return pl.pallas_call(
        _tc2_body,
        grid=(NPAD // bn,),
        in_specs=[
            pl.BlockSpec((2, bn, D), lambda i: (0, i, 0)),
            pl.BlockSpec((2, bn), lambda i: (0, i)),
            pl.BlockSpec((R, D, D), lambda i: (0, 0, 0)),
        ],
        out_specs=pl.BlockSpec((R, bn, D), lambda i: (0, i, 0)),
        out_shape=jax.ShapeDtypeStruct((R, NPAD, D), jnp.float32),
    )(acc1, cnt, W2)


def _tc3(acc2, cnt):
    bn = 2048
    return pl.pallas_call(
        _tc3_body,
        grid=(NPAD // bn,),
        in_specs=[
            pl.BlockSpec((2, bn, D), lambda i: (0, i, 0)),
            pl.BlockSpec((2, bn), lambda i: (0, i)),
        ],
        out_specs=pl.BlockSpec((bn, T), lambda i: (i, 0)),
        out_shape=jax.ShapeDtypeStruct((NPAD, T), jnp.float32),
    )(acc2, cnt)


_sc1 = _sc_edge_pass(D, use_nid_table=True)
_sc2 = _sc_edge_pass(D, use_nid_table=False)


def kernel(node_ids, edge_index, etype, entity, W1, W2):
    node_ids = node_ids.astype(jnp.int32)
    src = edge_index[0].astype(jnp.int32)
    dst = edge_index[1].astype(jnp.int32)
    etype = etype.astype(jnp.int32)
    pad = E_PAD - E
    shp = (NTILES * N_CHUNKS, CHUNK)
    srcp = jnp.concatenate([src, jnp.zeros((pad,), jnp.int32)]).reshape(shp)
    dstp = jnp.concatenate([dst, jnp.full((pad,), NPAD - 1, jnp.int32)]).reshape(shp)
    etyp = jnp.concatenate([etype, jnp.zeros((pad,), jnp.int32)]).reshape(shp)
    zrow = jnp.zeros((ROWS_PER_SUB, D), jnp.float32)
    zcnt = jnp.zeros((ROWS_PER_SUB,), jnp.float32)
    W2p = jnp.pad(W2, ((0, 0), (0, 0), (0, D - T)))

    ew1 = _tc1(entity, W1).reshape(R * N, D)
    acc1, cnt = _sc1(node_ids, srcp, dstp, etyp, ew1, zrow, zcnt)
    ew2 = _tc2(acc1, cnt, W2p).reshape(R * NPAD, D)
    acc2 = _sc2(srcp, dstp, etyp, ew2, zrow)
    y = _tc3(acc2, cnt)[:N]
    return (y[None], jnp.ones((1, N), jnp.float32))


# speedup vs baseline: 1.1549x; 1.1549x over previous
import functools

import jax
import jax.numpy as jnp
from jax import lax
from jax.experimental import pallas as pl
from jax.experimental.pallas import tpu as pltpu
from jax.experimental.pallas import tpu_sc as plsc

N = 10000
E = 320000
D = 128
R = 8
T = 64
NTILES = 32
NPAD = 10240
CHUNK = 64
N_CHUNKS = 160
NC0 = 240
NC1 = 2 * N_CHUNKS - NC0
EPT = N_CHUNKS * CHUNK
E_PAD = NTILES * EPT
ROWS_PER_SUB = NPAD // 16
QC = 16
NBUF = 4
LANES = 16

_MESH = plsc.VectorSubcoreMesh(core_axis_name="c", subcore_axis_name="s")


def _sc_edge_pass(msg_width, use_nid_table):
    stride = N if use_nid_table else NPAD

    def body(*refs):
        if use_nid_table:
            (nid_hbm, src_hbm, dst_hbm, ety_hbm, tab_hbm, zrow_hbm, zcnt_hbm,
             acc_hbm, cnt_hbm,
             srcbuf, etybuf, dstbuf, rowbuf, mb0, mb1, mb2, mb3, onesbuf,
             acc_sh, cnt_sh, sem_a, sg0, sg1, sg2, sg3, ss0, ss1, ss2,
             ss3) = refs
        else:
            (src_hbm, dst_hbm, ety_hbm, tab_hbm, zrow_hbm,
             acc_hbm,
             srcbuf, etybuf, dstbuf, rowbuf, mb0, mb1, mb2, mb3,
             acc_sh, sem_a, sg0, sg1, sg2, sg3, ss0, ss1, ss2, ss3) = refs
        c = lax.axis_index("c")
        s = lax.axis_index("s")
        gwid = c * 16 + s

        pltpu.sync_copy(zrow_hbm, acc_sh.at[pl.ds(s * ROWS_PER_SUB, ROWS_PER_SUB)])
        if use_nid_table:
            pltpu.sync_copy(zcnt_hbm, cnt_sh.at[pl.ds(s * ROWS_PER_SUB, ROWS_PER_SUB)])
            for i in range(CHUNK // LANES):
                onesbuf[pl.ds(i * LANES, LANES)] = jnp.ones((LANES,), jnp.float32)
        plsc.subcore_barrier()

        bufs = (mb0, mb1, mb2, mb3)
        gsems = (sg0, sg1, sg2, sg3)
        ssems = (ss0, ss1, ss2, ss3)
        nsp = jnp.where(c == 0, NC0 // QC, NC1 // QC)
        base_chunk = jnp.where(c == 0, s * NC0, 16 * NC0 + s * NC1)

        def subpass(q, carry0):
            qb = pl.ds(base_chunk + q * QC, QC)
            cps = [pltpu.async_copy(src_hbm.at[qb], srcbuf, sem_a),
                   pltpu.async_copy(ety_hbm.at[qb], etybuf, sem_a),
                   pltpu.async_copy(dst_hbm.at[qb], dstbuf, sem_a)]
            for cp in cps:
                cp.wait()
            if use_nid_table:
                def nid_fire(j, carry):
                    pltpu.async_copy(nid_hbm.at[srcbuf.at[j]], rowbuf.at[j],
                                     sem_a)
                    return carry

                lax.fori_loop(0, QC, nid_fire, 0)

                def nid_drain(j, carry):
                    pltpu.make_async_copy(nid_hbm.at[srcbuf.at[0]],
                                          rowbuf.at[0], sem_a).wait()
                    return carry

                lax.fori_loop(0, QC, nid_drain, 0)

            def rowcalc(j, carry):
                for i in range(CHUNK // LANES):
                    sl = pl.ds(i * LANES, LANES)
                    s16 = rowbuf[j, sl] if use_nid_table else srcbuf[j, sl]
                    rowbuf[j, sl] = etybuf[j, sl] * stride + s16
                return carry

            lax.fori_loop(0, QC, rowcalc, 0)

            for b in range(NBUF):
                pltpu.async_copy(tab_hbm.at[rowbuf.at[b]], bufs[b], gsems[b])

            def chunk_quad(jj, carry):
                for b in range(NBUF):
                    j = NBUF * jj + b
                    pltpu.make_async_copy(tab_hbm.at[rowbuf.at[0]], bufs[b],
                                          gsems[b]).wait()
                    cp_m = pltpu.async_copy(bufs[b], acc_sh.at[dstbuf.at[j]],
                                            ssems[b], add=True)
                    if use_nid_table:
                        cp_c = pltpu.async_copy(onesbuf,
                                                cnt_sh.at[dstbuf.at[j]],
                                                ssems[b], add=True)
                    cp_m.wait()
                    if use_nid_table:
                        cp_c.wait()

                    @pl.when(jj < QC // NBUF - 1)
                    def _():
                        pltpu.async_copy(tab_hbm.at[rowbuf.at[j + NBUF]],
                                         bufs[b], gsems[b])
                return carry

            lax.fori_loop(0, QC // NBUF, chunk_quad, 0)
            return carry0

        lax.fori_loop(0, nsp, subpass, 0)
        plsc.subcore_barrier()

        rows = pl.ds(s * ROWS_PER_SUB, ROWS_PER_SUB)
        pltpu.sync_copy(acc_sh.at[rows], acc_hbm.at[c, rows])
        if use_nid_table:
            pltpu.sync_copy(cnt_sh.at[rows], cnt_hbm.at[c, rows])

    if use_nid_table:
        out_type = (jax.ShapeDtypeStruct((2, NPAD, msg_width), jnp.float32),
                    jax.ShapeDtypeStruct((2, NPAD), jnp.float32))
        scratch = [
            pltpu.VMEM((QC, CHUNK), jnp.int32),
            pltpu.VMEM((QC, CHUNK), jnp.int32),
            pltpu.VMEM((QC, CHUNK), jnp.int32),
            pltpu.VMEM((QC, CHUNK), jnp.int32),
            pltpu.VMEM((CHUNK, msg_width), jnp.float32),
            pltpu.VMEM((CHUNK, msg_width), jnp.float32),
            pltpu.VMEM((CHUNK, msg_width), jnp.float32),
            pltpu.VMEM((CHUNK, msg_width), jnp.float32),
            pltpu.VMEM((CHUNK,), jnp.float32),
            pltpu.VMEM_SHARED((NPAD, msg_width), jnp.float32),
            pltpu.VMEM_SHARED((NPAD,), jnp.float32),
        ] + [pltpu.SemaphoreType.DMA] * 9
    else:
        out_type = jax.ShapeDtypeStruct((2, NPAD, msg_width), jnp.float32)
        scratch = [
            pltpu.VMEM((QC, CHUNK), jnp.int32),
            pltpu.VMEM((QC, CHUNK), jnp.int32),
            pltpu.VMEM((QC, CHUNK), jnp.int32),
            pltpu.VMEM((QC, CHUNK), jnp.int32),
            pltpu.VMEM((CHUNK, msg_width), jnp.float32),
            pltpu.VMEM((CHUNK, msg_width), jnp.float32),
            pltpu.VMEM((CHUNK, msg_width), jnp.float32),
            pltpu.VMEM((CHUNK, msg_width), jnp.float32),
            pltpu.VMEM_SHARED((NPAD, msg_width), jnp.float32),
        ] + [pltpu.SemaphoreType.DMA] * 9
    return pl.kernel(body, mesh=_MESH, out_type=out_type, scratch_types=scratch)


def _dot(a, b):
    return lax.dot_general(a, b, (((1,), (0,)), ((), ())),
                           preferred_element_type=jnp.float32,
                           precision=lax.Precision.HIGHEST)


def _tc1_body(x_ref, w_ref, o_ref):
    x = x_ref[...]
    for r in range(R):
        o_ref[r] = _dot(x, w_ref[r])


def _tc2_body(a_ref, c_ref, w_ref, o_ref):
    cnt = jnp.maximum(c_ref[0] + c_ref[1], 1.0)
    h = jnp.maximum((a_ref[0] + a_ref[1]) / cnt[:, None], 0.0)
    for r in range(R):
        o_ref[r] = _dot(h, w_ref[r])


def _tc3_body(a_ref, c_ref, o_ref):
    cnt = jnp.maximum(c_ref[0] + c_ref[1], 1.0)
    o_ref[...] = ((a_ref[0] + a_ref[1]) / cnt[:, None])[:, :T]


def _tc1(entity, W1):
    bn = 2000
    return pl.pallas_call(
        _tc1_body,
        grid=(N // bn,),
        in_specs=[
            pl.BlockSpec((bn, D), lambda i: (i, 0)),
            pl.BlockSpec((R, D, D), lambda i: (0, 0, 0)),
        ],
        out_specs=pl.BlockSpec((R, bn, D), lambda i: (0, i, 0)),
        out_shape=jax.ShapeDtypeStruct((R, N, D), jnp.float32),
    )(entity, W1)


def _tc2(acc1, cnt, W2):
    bn = 2048
    return pl.pallas_call(
        _tc2_body,
        grid=(NPAD // bn,),
        in_specs=[
            pl.BlockSpec((2, bn, D), lambda i: (0, i, 0)),
            pl.BlockSpec((2, bn), lambda i: (0, i)),
            pl.BlockSpec((R, D, D), lambda i: (0, 0, 0)),
        ],
        out_specs=pl.BlockSpec((R, bn, D), lambda i: (0, i, 0)),
        out_shape=jax.ShapeDtypeStruct((R, NPAD, D), jnp.float32),
    )(acc1, cnt, W2)


def _tc3(acc2, cnt):
    bn = 2048
    return pl.pallas_call(
        _tc3_body,
        grid=(NPAD // bn,),
        in_specs=[
            pl.BlockSpec((2, bn, D), lambda i: (0, i, 0)),
            pl.BlockSpec((2, bn), lambda i: (0, i)),
        ],
        out_specs=pl.BlockSpec((bn, T), lambda i: (i, 0)),
        out_shape=jax.ShapeDtypeStruct((NPAD, T), jnp.float32),
    )(acc2, cnt)


_sc1 = _sc_edge_pass(D, use_nid_table=True)
_sc2 = _sc_edge_pass(D, use_nid_table=False)


def kernel(node_ids, edge_index, etype, entity, W1, W2):
    node_ids = node_ids.astype(jnp.int32)
    src = edge_index[0].astype(jnp.int32)
    dst = edge_index[1].astype(jnp.int32)
    etype = etype.astype(jnp.int32)
    pad = E_PAD - E
    shp = (NTILES * N_CHUNKS, CHUNK)
    srcp = jnp.concatenate([src, jnp.zeros((pad,), jnp.int32)]).reshape(shp)
    dstp = jnp.concatenate([dst, jnp.full((pad,), NPAD - 1, jnp.int32)]).reshape(shp)
    etyp = jnp.concatenate([etype, jnp.zeros((pad,), jnp.int32)]).reshape(shp)
    zrow = jnp.zeros((ROWS_PER_SUB, D), jnp.float32)
    zcnt = jnp.zeros((ROWS_PER_SUB,), jnp.float32)
    W2p = jnp.pad(W2, ((0, 0), (0, 0), (0, D - T)))

    ew1 = _tc1(entity, W1).reshape(R * N, D)
    acc1, cnt = _sc1(node_ids, srcp, dstp, etyp, ew1, zrow, zcnt)
    ew2 = _tc2(acc1, cnt, W2p).reshape(R * NPAD, D)
    acc2 = _sc2(srcp, dstp, etyp, ew2, zrow)
    y = _tc3(acc2, cnt)[:N]
    return (y[None], jnp.ones((1, N), jnp.float32))
